# baseline (speedup 1.0000x reference)
import functools

import jax
import jax.numpy as jnp
from jax import lax
from jax.experimental import pallas as pl
from jax.experimental.pallas import tpu as pltpu
from jax.experimental.pallas import tpu_sc as plsc

_N = 10000
_E = 320000
_D = 128
_NC = 2
_NS = 16
_NW = _NC * _NS
_CHUNK = 80
_CHUNKS = 128
_EPAD = _NW * _CHUNKS * _CHUNK
_NP = 10112
_RPT = _NP // _NS


def _sc_aggregate(x_pad, pk_p, zeros_np):
    @functools.partial(
        pl.kernel,
        out_type=jax.ShapeDtypeStruct((_NC, _NP, _D), jnp.float32),
        mesh=plsc.VectorSubcoreMesh(core_axis_name="c", subcore_axis_name="s"),
        scratch_types=[
            pltpu.VMEM((_CHUNKS, _CHUNK), jnp.int32),
            pltpu.VMEM((_CHUNK,), jnp.int32),
            pltpu.VMEM((_CHUNK,), jnp.int32),
            pltpu.VMEM((_CHUNK,), jnp.int32),
            pltpu.VMEM((_CHUNK,), jnp.int32),
            pltpu.VMEM((_CHUNK, _D), jnp.float32),
            pltpu.VMEM((_CHUNK, _D), jnp.float32),
            pltpu.VMEM_SHARED((_NP, _D), jnp.float32),
            pltpu.SemaphoreType.DMA,
            pltpu.SemaphoreType.DMA,
            pltpu.SemaphoreType.DMA,
            pltpu.SemaphoreType.DMA,
        ],
    )
    def agg_kernel(x_hbm, pk_hbm, z_hbm, out_hbm,
                   pk_v, si0, si1, di0, di1, r0, r1, agg_s,
                   gs0, gs1, ss0, ss1):
        c = lax.axis_index("c")
        s = lax.axis_index("s")
        wid = c * _NS + s
        rows = (r0, r1)
        sidx = (si0, si1)
        didx = (di0, di1)
        gsem = (gs0, gs1)
        ssem = (ss0, ss1)

        def unpack(j, b):
            for q in range(_CHUNK // 16):
                v = pk_v[j, pl.ds(q * 16, 16)]
                sidx[b][pl.ds(q * 16, 16)] = v & 0xFFFF
                didx[b][pl.ds(q * 16, 16)] = v >> 16

        def gather(b):
            pltpu.async_copy(x_hbm.at[sidx[b]], rows[b], gsem[b])

        def gather_wait(b):
            pltpu.make_async_copy(x_hbm.at[sidx[b]], rows[b],
                                  gsem[b]).wait()

        def scatter(b):
            pltpu.async_copy(rows[b], agg_s.at[didx[b]], ssem[b], add=True)

        def scatter_wait(b):
            pltpu.make_async_copy(rows[b], agg_s.at[didx[b]],
                                  ssem[b]).wait()

        pltpu.sync_copy(pk_hbm.at[wid], pk_v)
        unpack(0, 0)
        gather(0)
        unpack(1, 1)
        gather(1)
        pltpu.sync_copy(z_hbm.at[pl.ds(s * _RPT, _RPT)],
                        agg_s.at[pl.ds(s * _RPT, _RPT)])
        plsc.subcore_barrier()

        def body(i, carry):
            j = 2 * i
            last = _CHUNKS - 1
            gather_wait(0)
            scatter(0)
            gather_wait(1)
            scatter(1)
            scatter_wait(0)
            unpack(jnp.minimum(j + 2, last), 0)
            gather(0)
            scatter_wait(1)
            unpack(jnp.minimum(j + 3, last), 1)
            gather(1)
            return carry

        lax.fori_loop(0, _CHUNKS // 2, body, 0)
        gather_wait(0)
        gather_wait(1)
        plsc.subcore_barrier()
        pltpu.sync_copy(agg_s.at[pl.ds(s * _RPT, _RPT)],
                        out_hbm.at[c, pl.ds(s * _RPT, _RPT)])

    return agg_kernel(x_pad, pk_p, zeros_np)


def _tc_linear(partials, x_pad, w_rel, w_root, b):
    def linear_body(p_ref, x_ref, wr_ref, wt_ref, b_ref, o_ref):
        agg = p_ref[0] + p_ref[1]
        o_ref[...] = (
            jnp.dot(agg, wr_ref[...], preferred_element_type=jnp.float32)
            + jnp.dot(x_ref[...], wt_ref[...], preferred_element_type=jnp.float32)
            + b_ref[...]
        )

    return pl.pallas_call(
        linear_body,
        out_shape=jax.ShapeDtypeStruct((_NP, _D), jnp.float32),
    )(partials, x_pad, w_rel, w_root, b.reshape(1, _D))


def kernel(edge_index, features, W1_rel, W1_root, b1, W2_rel, W2_root, b2):
    src = edge_index[0].astype(jnp.int32)
    dst = edge_index[1].astype(jnp.int32)
    pad = _EPAD - _E
    packed = src | (dst << 16)
    pk_p = jnp.concatenate([packed, jnp.full((pad,), _N | (_N << 16),
                                             jnp.int32)])
    pk_p = pk_p.reshape(_NW, _CHUNKS, _CHUNK)

    x_pad = jnp.zeros((_NP, _D), jnp.float32).at[:_N].set(features)
    zeros_np = jnp.zeros((_NP, _D), jnp.float32)

    p1 = _sc_aggregate(x_pad, pk_p, zeros_np)
    h_pad = _tc_linear(p1, x_pad, W1_rel, W1_root, b1)

    p2 = _sc_aggregate(h_pad, pk_p, zeros_np)
    out_pad = _tc_linear(p2, h_pad, W2_rel, W2_root, b2)

    return out_pad[:_N]
